# baseline (speedup 1.0000x reference)
import functools

import jax
import jax.numpy as jnp
from jax import lax
from jax.experimental import pallas as pl
from jax.experimental.pallas import tpu as pltpu
from jax.experimental.pallas import tpu_sc as plsc

N_VOX = 100000
IN_FEAT = 64
K_VOL = 27
CLS_OUT = 4
REG_OUT = 14
D_OUT = 18
D_PAD = 32

NW = 32
B = 128
N_PAD = 102400
CHUNKS = N_PAD // (NW * B)
PER_W = N_PAD // NW

BN = 2048
NI = N_PAD // BN


def _mm_body(x_ref, w_ref, b_ref, y_ref):
    y_ref[...] = (
        jnp.dot(x_ref[...], w_ref[0], preferred_element_type=jnp.float32)
        + b_ref[0, 0]
    )


def _tc_matmul(x_p, w_all, b_all):
    return pl.pallas_call(
        _mm_body,
        grid=(NI, K_VOL),
        in_specs=[
            pl.BlockSpec((BN, IN_FEAT), lambda i, k: (i, 0)),
            pl.BlockSpec((1, IN_FEAT, D_PAD), lambda i, k: (k, 0, 0)),
            pl.BlockSpec((1, 1, D_PAD), lambda i, k: (k, 0, 0)),
        ],
        out_specs=pl.BlockSpec((BN, D_PAD), lambda i, k: (k * NI + i, 0)),
        out_shape=jax.ShapeDtypeStruct((K_VOL * N_PAD, D_PAD), jnp.float32),
    )(x_p, w_all, b_all)


def _sc_body(y_hbm, idx_hbm, out_hbm, idxr_v, idx_v, buf_v, acc_v, sem):
    w = lax.axis_index("s") * 2 + lax.axis_index("c")
    iota = lax.iota(jnp.int32, 16)

    def chunk(c, carry):
        base = w * PER_W + c * B
        pltpu.sync_copy(idx_hbm.at[pl.ds(base, B)], idxr_v)
        for k in range(K_VOL):
            cols = jnp.full((16,), k, jnp.int32)
            for j in range(B // 16):
                v = plsc.load_gather(idxr_v, [iota + (j * 16), cols])
                idx_v[k, pl.ds(j * 16, 16)] = v + k * N_PAD
        copies = [
            pltpu.async_copy(y_hbm.at[idx_v.at[k]], buf_v.at[k], sem)
            for k in range(K_VOL)
        ]
        for cp in copies:
            cp.wait()

        def red(r, carry2):
            for h in (0, 16):
                v = buf_v[0, r, pl.ds(h, 16)]
                for k in range(1, K_VOL):
                    v = v + buf_v[k, r, pl.ds(h, 16)]
                acc_v[r, pl.ds(h, 16)] = v
            return carry2

        lax.fori_loop(0, B, red, 0)
        pltpu.sync_copy(acc_v, out_hbm.at[pl.ds(base, B)])
        return carry

    lax.fori_loop(0, CHUNKS, chunk, 0)


def _sc_gather_reduce(y, idx_r):
    mesh = plsc.VectorSubcoreMesh(core_axis_name="c", subcore_axis_name="s")
    fn = pl.kernel(
        _sc_body,
        out_type=jax.ShapeDtypeStruct((N_PAD, D_PAD), jnp.float32),
        mesh=mesh,
        scratch_types=[
            pltpu.VMEM((B, K_VOL), jnp.int32),
            pltpu.VMEM((K_VOL, B), jnp.int32),
            pltpu.VMEM((K_VOL, B, D_PAD), jnp.float32),
            pltpu.VMEM((B, D_PAD), jnp.float32),
            pltpu.SemaphoreType.DMA,
        ],
        compiler_params=pltpu.CompilerParams(
            use_tc_tiling_on_sc=False, needs_layout_passes=False
        ),
    )
    return fn(y, idx_r)


def kernel(x, neighbor_idx, W_cls, b_cls, W_reg, b_reg):
    x_p = jnp.pad(x, ((0, N_PAD - N_VOX), (0, 0)))
    w_all = jnp.concatenate([W_cls, W_reg], axis=2)
    w_all = jnp.pad(w_all, ((0, 0), (0, 0), (0, D_PAD - D_OUT)))
    b_all = jnp.concatenate([b_cls, b_reg])
    b_all = jnp.pad(b_all, (0, D_PAD - D_OUT))
    b_arr = jnp.zeros((K_VOL, 1, D_PAD), jnp.float32).at[0, 0].set(b_all)

    idx32 = neighbor_idx.astype(jnp.int32)
    idx_r = jnp.pad(idx32, ((0, N_PAD - N_VOX), (0, 0)))

    y = _tc_matmul(x_p, w_all, b_arr)

    out = _sc_gather_reduce(y, idx_r)

    return out[:N_VOX, :CLS_OUT], out[:N_VOX, CLS_OUT:D_OUT]


# speedup vs baseline: 2.2951x; 2.2951x over previous
import jax
import jax.numpy as jnp
from jax import lax
from jax.experimental import pallas as pl
from jax.experimental.pallas import tpu as pltpu
from jax.experimental.pallas import tpu_sc as plsc

N_VOX = 100000
IN_FEAT = 64
K_VOL = 27
CLS_OUT = 4
REG_OUT = 14
D_OUT = 18
D_PAD = 32
PACK = 4

NW = 32
B = 128
N_PAD = 102400
CHUNKS = N_PAD // (NW * B)
PER_W = N_PAD // NW

G = N_PAD // PACK
BG = 1024
NI = G // BG


def _mm_body(x_ref, w_ref, b_ref, y_ref):
    y_ref[...] = (
        jnp.dot(x_ref[...], w_ref[0], preferred_element_type=jnp.float32)
        + b_ref[0, 0]
    )


def _tc_matmul(x4, w4, b4):
    return pl.pallas_call(
        _mm_body,
        grid=(NI, K_VOL),
        in_specs=[
            pl.BlockSpec((BG, PACK * IN_FEAT), lambda i, k: (i, 0)),
            pl.BlockSpec((1, PACK * IN_FEAT, PACK * D_PAD), lambda i, k: (k, 0, 0)),
            pl.BlockSpec((1, 1, PACK * D_PAD), lambda i, k: (k, 0, 0)),
        ],
        out_specs=pl.BlockSpec((BG, PACK * D_PAD), lambda i, k: (k * NI + i, 0)),
        out_shape=jax.ShapeDtypeStruct((K_VOL * G, PACK * D_PAD), jnp.float32),
    )(x4, w4, b4)


def _sc_body(y_hbm, idx_hbm, out_hbm, idx_v, buf_v, acc_v, sem):
    w = lax.axis_index("s") * 2 + lax.axis_index("c")

    def chunk(c, carry):
        pltpu.sync_copy(idx_hbm.at[w, c], idx_v)
        copies = [
            pltpu.async_copy(y_hbm.at[idx_v.at[k]], buf_v.at[k], sem)
            for k in range(K_VOL)
        ]
        for cp in copies:
            cp.wait()

        def red(r, carry2):
            for h in (0, 16):
                v = buf_v[0, r, pl.ds(h, 16)]
                for k in range(1, K_VOL):
                    v = v + buf_v[k, r, pl.ds(h, 16)]
                acc_v[r, pl.ds(h, 16)] = v
            return carry2

        lax.fori_loop(0, B, red, 0)
        pltpu.sync_copy(acc_v, out_hbm.at[pl.ds(w * PER_W + c * B, B)])
        return carry

    lax.fori_loop(0, CHUNKS, chunk, 0)


def _sc_gather_reduce(y, idx_r):
    mesh = plsc.VectorSubcoreMesh(core_axis_name="c", subcore_axis_name="s")
    fn = pl.kernel(
        _sc_body,
        out_type=jax.ShapeDtypeStruct((N_PAD, D_PAD), jnp.float32),
        mesh=mesh,
        scratch_types=[
            pltpu.VMEM((K_VOL, B), jnp.int32),
            pltpu.VMEM((K_VOL, B, D_PAD), jnp.float32),
            pltpu.VMEM((B, D_PAD), jnp.float32),
            pltpu.SemaphoreType.DMA,
        ],
        compiler_params=pltpu.CompilerParams(use_tc_tiling_on_sc=False),
    )
    return fn(y, idx_r)


def kernel(x, neighbor_idx, W_cls, b_cls, W_reg, b_reg):
    x_p = jnp.pad(x, ((0, N_PAD - N_VOX), (0, 0)))
    x4 = x_p.reshape(G, PACK * IN_FEAT)

    w_all = jnp.concatenate([W_cls, W_reg], axis=2)
    w_all = jnp.pad(w_all, ((0, 0), (0, 0), (0, D_PAD - D_OUT)))
    eye4 = jnp.eye(PACK, dtype=jnp.float32)
    w4 = jax.vmap(lambda wk: jnp.kron(eye4, wk))(w_all)

    b_all = jnp.concatenate([b_cls, b_reg])
    b_all = jnp.pad(b_all, (0, D_PAD - D_OUT))
    b4 = jnp.zeros((K_VOL, 1, PACK * D_PAD), jnp.float32)
    b4 = b4.at[0, 0].set(jnp.tile(b_all, PACK))

    idx32 = neighbor_idx.astype(jnp.int32)
    idx_p = jnp.pad(idx32, ((0, N_PAD - N_VOX), (0, 0)))
    flat = idx_p + jnp.arange(K_VOL, dtype=jnp.int32)[None, :] * N_PAD
    idx_r = flat.reshape(NW, CHUNKS, B, K_VOL).transpose(0, 1, 3, 2)

    y4 = _tc_matmul(x4, w4, b4)
    y = y4.reshape(K_VOL * N_PAD, D_PAD)

    out = _sc_gather_reduce(y, idx_r)

    return out[:N_VOX, :CLS_OUT], out[:N_VOX, CLS_OUT:D_OUT]
